# baseline (speedup 1.0000x reference)
import functools

import jax
import jax.numpy as jnp
from jax import lax
from jax.experimental import pallas as pl
from jax.experimental.pallas import tpu as pltpu
from jax.experimental.pallas import tpu_sc as plsc

_B = 512
_C = 512
_L = 16
_NW = 32
_B_PER_W = _B // _NW
_ELEMS_W = _B_PER_W * _C
_CHUNK = 128
_NCHUNK = _ELEMS_W // _CHUNK


def _diag_body(x_hbm, out_hbm, idx_v, row_v, sem):
    cid = lax.axis_index("c")
    sid = lax.axis_index("s")
    wid = sid * 2 + cid
    lanes = lax.iota(jnp.int32, 16)

    for g in range(_NCHUNK):
        b = wid * _B_PER_W + g // 4
        c0 = (g % 4) * _CHUNK
        base = b * (_C * _C) + 513 * c0
        for h in range(_CHUNK // _L):
            idx_v[g, pl.ds(h * _L, _L)] = (base + 513 * (h * _L)) + 513 * lanes

    copies = []
    for g in range(_NCHUNK):
        copies.append(
            pltpu.async_copy(
                x_hbm.at[idx_v.at[g]],
                row_v.at[pl.ds(g * _CHUNK, _CHUNK)],
                sem,
            )
        )
    for cp in copies:
        cp.wait()

    pltpu.sync_copy(row_v, out_hbm.at[wid])


@jax.jit
def _diag(x1d):
    mesh = plsc.VectorSubcoreMesh(core_axis_name="c", subcore_axis_name="s")
    f = functools.partial(
        pl.kernel,
        mesh=mesh,
        out_type=jax.ShapeDtypeStruct((_NW, _ELEMS_W), jnp.float32),
        scratch_types=[
            pltpu.VMEM((_NCHUNK, _CHUNK), jnp.int32),
            pltpu.VMEM((_ELEMS_W,), jnp.float32),
            pltpu.SemaphoreType.DMA,
        ],
    )(_diag_body)
    return f(x1d)


def kernel(x):
    B, C, C1 = x.shape
    assert (B, C, C1) == (_B, _C, _C)
    return _diag(x.reshape(-1)).reshape(_B, _C)


# speedup vs baseline: 5.4388x; 5.4388x over previous
import functools

import jax
import jax.numpy as jnp
from jax import lax
from jax.experimental import pallas as pl
from jax.experimental.pallas import tpu as pltpu
from jax.experimental.pallas import tpu_sc as plsc

_B = 512
_C = 512
_L = 16
_T = 128
_NW = 32
_B_PER_W = _B // _NW
_BLK_PER_B = _C // _T
_BLK_PER_W = _B_PER_W * _BLK_PER_B


def _diag_body(x_hbm, out_hbm, buf_v, out_v, sem):
    cid = lax.axis_index("c")
    sid = lax.axis_index("s")
    wid = sid * 2 + cid
    lanes = lax.iota(jnp.int32, 16)

    def per_round(r, carry):
        g0 = (wid * _B_PER_W + r) * _BLK_PER_B
        copies = []
        for j in range(_BLK_PER_B):
            g = g0 + j
            row0 = pl.multiple_of(g * _T, _T)
            col0 = j * _T
            copies.append(
                pltpu.async_copy(
                    x_hbm.at[pl.ds(row0, _T), pl.ds(col0, _T)],
                    buf_v.at[j],
                    sem,
                )
            )
        for cp in copies:
            cp.wait()
        for j in range(_BLK_PER_B):
            for q in range(_T // _L):
                d = q * _L + lanes
                diag = plsc.load_gather(buf_v.at[j], [d, d])
                out_v[r, pl.ds(j * _T + q * _L, _L)] = diag
        return carry

    lax.fori_loop(0, _B_PER_W, per_round, 0)
    pltpu.sync_copy(out_v, out_hbm.at[pl.ds(wid * _B_PER_W, _B_PER_W)])


@jax.jit
def _diag(x2d):
    mesh = plsc.VectorSubcoreMesh(core_axis_name="c", subcore_axis_name="s")
    f = functools.partial(
        pl.kernel,
        mesh=mesh,
        out_type=jax.ShapeDtypeStruct((_B, _C), jnp.float32),
        scratch_types=[
            pltpu.VMEM((_BLK_PER_B, _T, _T), jnp.float32),
            pltpu.VMEM((_B_PER_W, _C), jnp.float32),
            pltpu.SemaphoreType.DMA,
        ],
        compiler_params=pltpu.CompilerParams(needs_layout_passes=False),
    )(_diag_body)
    return f(x2d)


def kernel(x):
    B, C, C1 = x.shape
    assert (B, C, C1) == (_B, _C, _C)
    return _diag(x.reshape(_B * _C, _C))


# speedup vs baseline: 5.4430x; 1.0008x over previous
import functools

import jax
import jax.numpy as jnp
from jax import lax
from jax.experimental import pallas as pl
from jax.experimental.pallas import tpu as pltpu
from jax.experimental.pallas import tpu_sc as plsc

_B = 512
_C = 512
_L = 16
_T = 128
_NW = 32
_B_PER_W = _B // _NW
_BLK_PER_B = _C // _T
_BLK_PER_W = _B_PER_W * _BLK_PER_B
_RB = 2
_NROUND = _BLK_PER_W // _RB


def _diag_body(x_hbm, out_hbm, buf_v, out_v, sem0, sem1):
    cid = lax.axis_index("c")
    sid = lax.axis_index("s")
    wid = sid * 2 + cid
    lanes = lax.iota(jnp.int32, 16)
    sems = [sem0, sem1]

    def block_refs(r, parity, k):
        g = (wid * _NROUND + r) * _RB + k
        row0 = pl.multiple_of(g * _T, _T)
        col0 = pl.multiple_of((g % _BLK_PER_B) * _T, _T)
        return x_hbm.at[pl.ds(row0, _T), pl.ds(col0, _T)], buf_v.at[parity, k]

    def fire(r, parity):
        for k in range(_RB):
            src, dst = block_refs(r, parity, k)
            pltpu.async_copy(src, dst, sems[parity])

    def drain_extract(r, parity, i, j0):
        for k in range(_RB):
            src, dst = block_refs(r, parity, k)
            pltpu.make_async_copy(src, dst, sems[parity]).wait()
        for k in range(_RB):
            for q in range(_T // _L):
                d = q * _L + lanes
                diag = plsc.load_gather(buf_v.at[parity, k], [d, d])
                out_v[i, pl.ds((j0 + k) * _T + q * _L, _L)] = diag

    def per_super(s, carry):
        fire(2 * s + 1, 1)
        drain_extract(2 * s, 0, s, 0)

        @pl.when(s + 1 < _NROUND // 2)
        def _():
            fire(2 * s + 2, 0)

        drain_extract(2 * s + 1, 1, s, 2)
        return carry

    fire(0, 0)
    lax.fori_loop(0, _NROUND // 2, per_super, 0)
    pltpu.sync_copy(out_v, out_hbm.at[pl.ds(wid * _B_PER_W, _B_PER_W)])


@jax.jit
def _diag(x2d):
    mesh = plsc.VectorSubcoreMesh(core_axis_name="c", subcore_axis_name="s")
    f = functools.partial(
        pl.kernel,
        mesh=mesh,
        out_type=jax.ShapeDtypeStruct((_B, _C), jnp.float32),
        scratch_types=[
            pltpu.VMEM((2, _RB, _T, _T), jnp.float32),
            pltpu.VMEM((_B_PER_W, _C), jnp.float32),
            pltpu.SemaphoreType.DMA,
            pltpu.SemaphoreType.DMA,
        ],
        compiler_params=pltpu.CompilerParams(needs_layout_passes=False),
    )(_diag_body)
    return f(x2d)


def kernel(x):
    B, C, C1 = x.shape
    assert (B, C, C1) == (_B, _C, _C)
    return _diag(x.reshape(_B * _C, _C))
